# baseline (speedup 1.0000x reference)
import functools

import jax
import jax.numpy as jnp
from jax import lax
from jax.experimental import pallas as pl
from jax.experimental.pallas import tpu as pltpu
from jax.experimental.pallas import tpu_sc as plsc

N_NODES = 10000
N_EDGES = 160000
D_FEAT = 256

_NC = 2
_NS = 16
_NW = _NC * _NS
_BPW = N_EDGES // _NW
_CH = 200
_NCH = 26
_LASTOFF = _BPW - _CH


def _gather_body(table, idx_vi, idx_vj, out_vi, out_vj,
                 idxv, rows0, rows1, gsem0, gsem1, ssem0, ssem1):
    wid = lax.axis_index("s") * _NC + lax.axis_index("c")
    base = wid * _BPW
    rows = (rows0, rows1)
    gsem = (gsem0, gsem1)
    ssem = (ssem0, ssem1)

    def off_of(i):
        return lax.min(i * _CH, _LASTOFF)

    for idx_hbm, out_hbm in ((idx_vi, out_vi), (idx_vj, out_vj)):
        pltpu.sync_copy(idx_hbm.at[pl.ds(base, _BPW)], idxv)
        pltpu.async_copy(table.at[idxv.at[pl.ds(0, _CH)]], rows0, gsem0)

        def body(g, carry, out_hbm=out_hbm):
            for b in (0, 1):
                i = 2 * g + b
                off = off_of(i)
                pltpu.make_async_copy(
                    table.at[idxv.at[pl.ds(off, _CH)]], rows[b], gsem[b]).wait()
                pltpu.async_copy(
                    rows[b], out_hbm.at[pl.ds(base + off, _CH)], ssem[b])

                @pl.when(i >= 1)
                def _(b=b):
                    pltpu.make_async_copy(
                        rows[1 - b], out_hbm.at[pl.ds(base, _CH)],
                        ssem[1 - b]).wait()

                @pl.when(i < _NCH - 1)
                def _(i=i, b=b):
                    noff = off_of(i + 1)
                    pltpu.async_copy(
                        table.at[idxv.at[pl.ds(noff, _CH)]], rows[1 - b],
                        gsem[1 - b])
            return carry

        lax.fori_loop(0, _NCH // 2, body, 0)
        lb = (_NCH - 1) % 2
        pltpu.make_async_copy(
            rows[lb], out_hbm.at[pl.ds(base, _CH)], ssem[lb]).wait()


_gather2 = functools.partial(
    pl.kernel,
    out_type=(
        jax.ShapeDtypeStruct((N_EDGES, D_FEAT), jnp.float32),
        jax.ShapeDtypeStruct((N_EDGES, D_FEAT), jnp.float32),
    ),
    mesh=plsc.VectorSubcoreMesh(core_axis_name="c", subcore_axis_name="s"),
    scratch_types=(
        pltpu.VMEM((_BPW,), jnp.int32),
        pltpu.VMEM((_CH, D_FEAT), jnp.float32),
        pltpu.VMEM((_CH, D_FEAT), jnp.float32),
        pltpu.SemaphoreType.DMA,
        pltpu.SemaphoreType.DMA,
        pltpu.SemaphoreType.DMA,
        pltpu.SemaphoreType.DMA,
    ),
)(_gather_body)


def kernel(inputs, selected_edges):
    idx_vi = selected_edges[:, 6]
    idx_vj = selected_edges[:, 7]
    return _gather2(inputs, idx_vi, idx_vj)


# speedup vs baseline: 1.0242x; 1.0242x over previous
import functools

import jax
import jax.numpy as jnp
from jax import lax
from jax.experimental import pallas as pl
from jax.experimental.pallas import tpu as pltpu
from jax.experimental.pallas import tpu_sc as plsc

N_NODES = 10000
N_EDGES = 160000
D_FEAT = 256

_NC = 2
_NS = 16
_NW = _NC * _NS
_BPW = N_EDGES // _NW
_CH = 112
_NCH = 46
_LASTOFF = _BPW - _CH


def _gather_body(table, idx_vi, idx_vj, out_vi, out_vj,
                 idxv0, idxv1, ra0, ra1, rb0, rb1,
                 ga0, ga1, gb0, gb1, sa0, sa1, sb0, sb1):
    wid = lax.axis_index("s") * _NC + lax.axis_index("c")
    base = wid * _BPW
    rings = (
        (idxv0, out_vi, (ra0, ra1), (ga0, ga1), (sa0, sa1)),
        (idxv1, out_vj, (rb0, rb1), (gb0, gb1), (sb0, sb1)),
    )
    pltpu.sync_copy(idx_vi.at[pl.ds(base, _BPW)], idxv0)
    pltpu.sync_copy(idx_vj.at[pl.ds(base, _BPW)], idxv1)
    for idxv, _out, rows, gsem, _ssem in rings:
        pltpu.async_copy(table.at[idxv.at[pl.ds(0, _CH)]], rows[0], gsem[0])

    def body(g, carry):
        for b in (0, 1):
            i = 2 * g + b
            off = lax.min(i * _CH, _LASTOFF)
            for idxv, out_hbm, rows, gsem, ssem in rings:
                pltpu.make_async_copy(
                    table.at[idxv.at[pl.ds(off, _CH)]], rows[b], gsem[b]).wait()
                pltpu.async_copy(
                    rows[b], out_hbm.at[pl.ds(base + off, _CH)], ssem[b])

                @pl.when(i >= 1)
                def _(rows=rows, out_hbm=out_hbm, ssem=ssem, b=b):
                    pltpu.make_async_copy(
                        rows[1 - b], out_hbm.at[pl.ds(base, _CH)],
                        ssem[1 - b]).wait()

                @pl.when(i < _NCH - 1)
                def _(idxv=idxv, rows=rows, gsem=gsem, i=i, b=b):
                    noff = lax.min((i + 1) * _CH, _LASTOFF)
                    pltpu.async_copy(
                        table.at[idxv.at[pl.ds(noff, _CH)]], rows[1 - b],
                        gsem[1 - b])
        return carry

    lax.fori_loop(0, _NCH // 2, body, 0)
    lb = (_NCH - 1) % 2
    for _idxv, out_hbm, rows, _gsem, ssem in rings:
        pltpu.make_async_copy(
            rows[lb], out_hbm.at[pl.ds(base, _CH)], ssem[lb]).wait()


_gather2 = functools.partial(
    pl.kernel,
    out_type=(
        jax.ShapeDtypeStruct((N_EDGES, D_FEAT), jnp.float32),
        jax.ShapeDtypeStruct((N_EDGES, D_FEAT), jnp.float32),
    ),
    mesh=plsc.VectorSubcoreMesh(core_axis_name="c", subcore_axis_name="s"),
    scratch_types=(
        pltpu.VMEM((_BPW,), jnp.int32),
        pltpu.VMEM((_BPW,), jnp.int32),
        pltpu.VMEM((_CH, D_FEAT), jnp.float32),
        pltpu.VMEM((_CH, D_FEAT), jnp.float32),
        pltpu.VMEM((_CH, D_FEAT), jnp.float32),
        pltpu.VMEM((_CH, D_FEAT), jnp.float32),
        pltpu.SemaphoreType.DMA,
        pltpu.SemaphoreType.DMA,
        pltpu.SemaphoreType.DMA,
        pltpu.SemaphoreType.DMA,
        pltpu.SemaphoreType.DMA,
        pltpu.SemaphoreType.DMA,
        pltpu.SemaphoreType.DMA,
        pltpu.SemaphoreType.DMA,
    ),
)(_gather_body)


def kernel(inputs, selected_edges):
    idx_vi = selected_edges[:, 6]
    idx_vj = selected_edges[:, 7]
    return _gather2(inputs, idx_vi, idx_vj)
